# baseline (speedup 1.0000x reference)
import jax
import jax.numpy as jnp
from jax import lax
from jax.experimental import pallas as pl
from jax.experimental.pallas import tpu as pltpu
from jax.experimental.pallas import tpu_sc as plsc

N = 10000
NP = 10240
DIN = 128
HID = 128
LAT = 64
E = 320000
NC, NS = 2, 16
NW = NC * NS
CH = 128
KCH = 80
KCH1 = 160
EP = NW * KCH * CH
PAD = NP - N
RPT = NP // NS

_MESH = plsc.VectorSubcoreMesh(
    core_axis_name="c", subcore_axis_name="s", num_cores=NC, num_subcores=NS)

_SC_LINEAR = pltpu.CompilerParams(use_tc_tiling_on_sc=False)


def _deg_body(dstp, zeros_r, ones_r, out, idx_v, zb, ones_v, acc):
    c = lax.axis_index("c")
    s = lax.axis_index("s")
    wid = c * NS + s
    pltpu.sync_copy(zeros_r, zb)
    pltpu.sync_copy(ones_r, ones_v)
    pltpu.sync_copy(zb, acc.at[pl.ds(s * RPT, RPT)])
    plsc.subcore_barrier()
    pltpu.sync_copy(dstp.at[wid], idx_v)

    def body(j, carry):
        pltpu.sync_copy(ones_v, acc.at[idx_v.at[j]], add=True)
        return carry

    lax.fori_loop(0, KCH, body, 0)
    plsc.subcore_barrier()
    pltpu.sync_copy(acc.at[pl.ds(s * RPT, RPT)], zb)
    pltpu.sync_copy(zb, out.at[c, pl.ds(s * RPT, RPT)])


_deg_call = pl.kernel(
    _deg_body,
    out_type=jax.ShapeDtypeStruct((NC, NP), jnp.float32),
    mesh=_MESH,
    compiler_params=_SC_LINEAR,
    scratch_types=[
        pltpu.VMEM((KCH, CH), jnp.int32),
        pltpu.VMEM((RPT,), jnp.float32),
        pltpu.VMEM((CH,), jnp.float32),
        pltpu.VMEM_SHARED((NP,), jnp.float32),
    ],
)


_NBUF = 4


def _gather_scatter_loop(y, src_v, dst_v, rows, acc, gsems, ssems, kch):
    for b in range(_NBUF):
        pltpu.async_copy(y.at[src_v.at[b]], rows[b], gsems[b])

    def body(p, carry):
        j0 = _NBUF * p
        for b in range(_NBUF):
            pltpu.make_async_copy(y.at[src_v.at[j0 + b]], rows[b],
                                  gsems[b]).wait()
            pltpu.async_copy(rows[b], acc.at[dst_v.at[j0 + b]], ssems[b],
                             add=True)
        for b in range(_NBUF):
            pltpu.make_async_copy(rows[b], acc.at[dst_v.at[j0 + b]],
                                  ssems[b]).wait()
            jn = jnp.minimum(j0 + _NBUF + b, kch - 1)
            pltpu.async_copy(y.at[src_v.at[jn]], rows[b], gsems[b])
        return carry

    lax.fori_loop(0, kch // _NBUF, body, 0)
    for b in range(_NBUF):
        pltpu.make_async_copy(y.at[src_v.at[kch - 1]], rows[b],
                              gsems[b]).wait()


def _zero_acc_slice(zrows, rows_a, acc, s):
    pltpu.sync_copy(zrows, rows_a)
    for k in range(RPT // CH):
        pltpu.sync_copy(rows_a, acc.at[pl.ds(s * RPT + k * CH, CH)])


def _writeback(acc, rows_a, out, c, s):
    for k in range(RPT // CH):
        off = s * RPT + k * CH
        pltpu.sync_copy(acc.at[pl.ds(off, CH)], rows_a)
        pltpu.sync_copy(rows_a, out.at[c, pl.ds(off, CH)])


def _agg_scratch(kch):
    return ([pltpu.VMEM((kch, CH), jnp.int32)] * 2
            + [pltpu.VMEM((CH, LAT), jnp.float32)] * _NBUF
            + [pltpu.VMEM_SHARED((NP, LAT), jnp.float32)]
            + [pltpu.SemaphoreType.DMA] * (2 * _NBUF))


def _agg_cs_body(ycs, srcp, dstp, zrows, out, src_v, dst_v, *scr):
    rows, acc = scr[:_NBUF], scr[_NBUF]
    gsems, ssems = scr[_NBUF + 1:2 * _NBUF + 1], scr[2 * _NBUF + 1:]
    c = lax.axis_index("c")
    s = lax.axis_index("s")
    _zero_acc_slice(zrows, rows[0], acc, s)
    plsc.subcore_barrier()
    pltpu.sync_copy(srcp.at[s], src_v)
    pltpu.sync_copy(dstp.at[s], dst_v)
    _gather_scatter_loop(ycs.at[c], src_v, dst_v, rows, acc, gsems, ssems,
                         KCH1)
    plsc.subcore_barrier()
    _writeback(acc, rows[0], out, c, s)


_agg_cs_call = pl.kernel(
    _agg_cs_body,
    out_type=jax.ShapeDtypeStruct((NC, NP, LAT), jnp.float32),
    mesh=_MESH,
    compiler_params=_SC_LINEAR,
    scratch_types=_agg_scratch(KCH1),
)


def _agg_es_body(y, srcp, dstp, zrows, out, src_v, dst_v, *scr):
    rows, acc = scr[:_NBUF], scr[_NBUF]
    gsems, ssems = scr[_NBUF + 1:2 * _NBUF + 1], scr[2 * _NBUF + 1:]
    c = lax.axis_index("c")
    s = lax.axis_index("s")
    wid = c * NS + s
    _zero_acc_slice(zrows, rows[0], acc, s)
    plsc.subcore_barrier()
    pltpu.sync_copy(srcp.at[wid], src_v)
    pltpu.sync_copy(dstp.at[wid], dst_v)
    _gather_scatter_loop(y, src_v, dst_v, rows, acc, gsems, ssems, KCH)
    plsc.subcore_barrier()
    _writeback(acc, rows[0], out, c, s)


_agg_es_call = pl.kernel(
    _agg_es_body,
    out_type=jax.ShapeDtypeStruct((NC, NP, LAT), jnp.float32),
    mesh=_MESH,
    compiler_params=_SC_LINEAR,
    scratch_types=_agg_scratch(KCH),
)


_BR = 1000


def _dinv_block(degt_ref):
    return lax.rsqrt(degt_ref[:, 0] + degt_ref[:, 1] + 1.0)


def _mm_scale_body(x_ref, w_ref, degp_ref, y_ref):
    dinv = _dinv_block(degp_ref)
    y = jnp.dot(x_ref[...], w_ref[...], preferred_element_type=jnp.float32)
    y = y * dinv[:, None]
    y_ref[0] = y[:, :LAT]
    y_ref[1] = y[:, LAT:]


_mm_scale_call = pl.pallas_call(
    _mm_scale_body,
    grid=(N // _BR,),
    in_specs=[
        pl.BlockSpec((_BR, DIN), lambda i: (i, 0)),
        pl.BlockSpec((DIN, HID), lambda i: (0, 0)),
        pl.BlockSpec((_BR, NC), lambda i: (i, 0)),
    ],
    out_specs=pl.BlockSpec((NC, _BR, LAT), lambda i: (0, i, 0)),
    out_shape=jax.ShapeDtypeStruct((NC, NP, LAT), jnp.float32),
)


def _fuse1_body(agg_ref, y1_ref, degp_ref, b1_ref, w2_ref, y2_ref):
    dinv = _dinv_block(degp_ref)
    a = jnp.concatenate([agg_ref[0] + y1_ref[0], agg_ref[1] + y1_ref[1]],
                        axis=1)
    z = jnp.maximum(a * dinv[:, None] + b1_ref[...][None, :], 0.0)
    y2 = jnp.dot(z, w2_ref[...], preferred_element_type=jnp.float32)
    y2_ref[...] = y2 * dinv[:, None]


_fuse1_call = pl.pallas_call(
    _fuse1_body,
    grid=(N // _BR,),
    in_specs=[
        pl.BlockSpec((NC, _BR, LAT), lambda i: (0, i, 0)),
        pl.BlockSpec((NC, _BR, LAT), lambda i: (0, i, 0)),
        pl.BlockSpec((_BR, NC), lambda i: (i, 0)),
        pl.BlockSpec((HID,), lambda i: (0,)),
        pl.BlockSpec((HID, LAT), lambda i: (0, 0)),
    ],
    out_specs=pl.BlockSpec((_BR, LAT), lambda i: (i, 0)),
    out_shape=jax.ShapeDtypeStruct((NP, LAT), jnp.float32),
)


def _fuse2_body(agg_ref, y2_ref, degp_ref, b2_ref, wmu_ref, bmu_ref,
                wlv_ref, blv_ref, mu_ref, lv_ref):
    dinv = _dinv_block(degp_ref)
    a = agg_ref[0] + agg_ref[1] + y2_ref[...]
    z = a * dinv[:, None] + b2_ref[...][None, :]
    mu_ref[...] = (jnp.dot(z, wmu_ref[...], preferred_element_type=jnp.float32)
                   + bmu_ref[...][None, :])
    lv_ref[...] = (jnp.dot(z, wlv_ref[...], preferred_element_type=jnp.float32)
                   + blv_ref[...][None, :])


_fuse2_call = pl.pallas_call(
    _fuse2_body,
    grid=(N // _BR,),
    in_specs=[
        pl.BlockSpec((NC, _BR, LAT), lambda i: (0, i, 0)),
        pl.BlockSpec((_BR, LAT), lambda i: (i, 0)),
        pl.BlockSpec((_BR, NC), lambda i: (i, 0)),
        pl.BlockSpec((LAT,), lambda i: (0,)),
        pl.BlockSpec((LAT, LAT), lambda i: (0, 0)),
        pl.BlockSpec((LAT,), lambda i: (0,)),
        pl.BlockSpec((LAT, LAT), lambda i: (0, 0)),
        pl.BlockSpec((LAT,), lambda i: (0,)),
    ],
    out_specs=[
        pl.BlockSpec((_BR, LAT), lambda i: (i, 0)),
        pl.BlockSpec((_BR, LAT), lambda i: (i, 0)),
    ],
    out_shape=[
        jax.ShapeDtypeStruct((N, LAT), jnp.float32),
        jax.ShapeDtypeStruct((N, LAT), jnp.float32),
    ],
)


def kernel(x, edge_index, W1, b1, W2, b2, W_mu, b_mu, W_lv, b_lv):
    pad_idx = (jnp.arange(EP - E, dtype=jnp.int32) % PAD) + N
    src = jnp.concatenate([edge_index[0], pad_idx])
    dst = jnp.concatenate([edge_index[1], pad_idx])
    srcp16 = src.reshape(NS, KCH1, CH)
    dstp16 = dst.reshape(NS, KCH1, CH)
    srcp32 = src.reshape(NW, KCH, CH)
    dstp32 = dst.reshape(NW, KCH, CH)
    zeros_r = jnp.zeros((RPT,), jnp.float32)
    ones_r = jnp.ones((CH,), jnp.float32)
    zrows = jnp.zeros((CH, LAT), jnp.float32)

    degp = _deg_call(dstp32, zeros_r, ones_r)
    degt = degp.T
    y1cs = _mm_scale_call(x, W1, degt)
    agg1 = _agg_cs_call(y1cs, srcp16, dstp16, zrows)
    y2 = _fuse1_call(agg1, y1cs, degt, b1, W2)
    agg2 = _agg_es_call(y2, srcp32, dstp32, zrows)
    mu, lv = _fuse2_call(agg2, y2, degt, b2, W_mu, b_mu, W_lv, b_lv)
    return (mu, lv)


# speedup vs baseline: 1.0292x; 1.0292x over previous
import jax
import jax.numpy as jnp
from jax import lax
from jax.experimental import pallas as pl
from jax.experimental.pallas import tpu as pltpu
from jax.experimental.pallas import tpu_sc as plsc

N = 10000
NP = 10240
DIN = 128
HID = 128
LAT = 64
E = 320000
NC, NS = 2, 16
NW = NC * NS
CH = 128
KCH = 80
KCH1 = 160
EP = NW * KCH * CH
PAD = NP - N
RPT = NP // NS

_MESH = plsc.VectorSubcoreMesh(
    core_axis_name="c", subcore_axis_name="s", num_cores=NC, num_subcores=NS)

_SC_LINEAR = pltpu.CompilerParams(use_tc_tiling_on_sc=False)


def _deg_body(dstp, zeros_r, ones_r, out, idx_v, zb, ones_v, acc):
    c = lax.axis_index("c")
    s = lax.axis_index("s")
    wid = c * NS + s
    pltpu.sync_copy(zeros_r, zb)
    pltpu.sync_copy(ones_r, ones_v)
    pltpu.sync_copy(zb, acc.at[pl.ds(s * RPT, RPT)])
    plsc.subcore_barrier()
    pltpu.sync_copy(dstp.at[wid], idx_v)

    def body(j, carry):
        pltpu.sync_copy(ones_v, acc.at[idx_v.at[j]], add=True)
        return carry

    lax.fori_loop(0, KCH, body, 0)
    plsc.subcore_barrier()
    pltpu.sync_copy(acc.at[pl.ds(s * RPT, RPT)], zb)
    pltpu.sync_copy(zb, out.at[c, pl.ds(s * RPT, RPT)])


_deg_call = pl.kernel(
    _deg_body,
    out_type=jax.ShapeDtypeStruct((NC, NP), jnp.float32),
    mesh=_MESH,
    compiler_params=_SC_LINEAR,
    scratch_types=[
        pltpu.VMEM((KCH, CH), jnp.int32),
        pltpu.VMEM((RPT,), jnp.float32),
        pltpu.VMEM((CH,), jnp.float32),
        pltpu.VMEM_SHARED((NP,), jnp.float32),
    ],
)


_NBUF = 6


def _gather_scatter_loop(y, src_v, dst_v, rows, acc, gsems, ssems, kch):
    for b in range(_NBUF):
        pltpu.async_copy(y.at[src_v.at[b]], rows[b], gsems[b])

    def body(p, carry):
        j0 = _NBUF * p
        for b in range(_NBUF):
            pltpu.make_async_copy(y.at[src_v.at[j0 + b]], rows[b],
                                  gsems[b]).wait()
            pltpu.async_copy(rows[b], acc.at[dst_v.at[j0 + b]], ssems[b],
                             add=True)
        for b in range(_NBUF):
            pltpu.make_async_copy(rows[b], acc.at[dst_v.at[j0 + b]],
                                  ssems[b]).wait()
            jn = jnp.minimum(j0 + _NBUF + b, kch - 1)
            pltpu.async_copy(y.at[src_v.at[jn]], rows[b], gsems[b])
        return carry

    lax.fori_loop(0, kch // _NBUF, body, 0)
    for b in range(_NBUF):
        pltpu.make_async_copy(y.at[src_v.at[kch - 1]], rows[b],
                              gsems[b]).wait()


def _zero_acc_slice(zrows, rows_a, acc, s):
    pltpu.sync_copy(zrows, rows_a)
    for k in range(RPT // CH):
        pltpu.sync_copy(rows_a, acc.at[pl.ds(s * RPT + k * CH, CH)])


def _writeback(acc, rows_a, out, c, s):
    for k in range(RPT // CH):
        off = s * RPT + k * CH
        pltpu.sync_copy(acc.at[pl.ds(off, CH)], rows_a)
        pltpu.sync_copy(rows_a, out.at[c, pl.ds(off, CH)])


def _agg_scratch(kch):
    return ([pltpu.VMEM((kch, CH), jnp.int32)] * 2
            + [pltpu.VMEM((CH, LAT), jnp.float32)] * _NBUF
            + [pltpu.VMEM_SHARED((NP, LAT), jnp.float32)]
            + [pltpu.SemaphoreType.DMA] * (2 * _NBUF))


def _agg_cs_body(ycs, srcp, dstp, zrows, out, src_v, dst_v, *scr):
    rows, acc = scr[:_NBUF], scr[_NBUF]
    gsems, ssems = scr[_NBUF + 1:2 * _NBUF + 1], scr[2 * _NBUF + 1:]
    c = lax.axis_index("c")
    s = lax.axis_index("s")
    _zero_acc_slice(zrows, rows[0], acc, s)
    plsc.subcore_barrier()
    pltpu.sync_copy(srcp.at[s], src_v)
    pltpu.sync_copy(dstp.at[s], dst_v)
    _gather_scatter_loop(ycs.at[c], src_v, dst_v, rows, acc, gsems, ssems,
                         KCH1)
    plsc.subcore_barrier()
    _writeback(acc, rows[0], out, c, s)


_agg_cs_call = pl.kernel(
    _agg_cs_body,
    out_type=jax.ShapeDtypeStruct((NC, NP, LAT), jnp.float32),
    mesh=_MESH,
    compiler_params=_SC_LINEAR,
    scratch_types=_agg_scratch(KCH1),
)


def _agg_es_body(y, srcp, dstp, zrows, out, src_v, dst_v, *scr):
    rows, acc = scr[:_NBUF], scr[_NBUF]
    gsems, ssems = scr[_NBUF + 1:2 * _NBUF + 1], scr[2 * _NBUF + 1:]
    c = lax.axis_index("c")
    s = lax.axis_index("s")
    wid = c * NS + s
    _zero_acc_slice(zrows, rows[0], acc, s)
    plsc.subcore_barrier()
    pltpu.sync_copy(srcp.at[wid], src_v)
    pltpu.sync_copy(dstp.at[wid], dst_v)
    _gather_scatter_loop(y, src_v, dst_v, rows, acc, gsems, ssems, KCH)
    plsc.subcore_barrier()
    _writeback(acc, rows[0], out, c, s)


_agg_es_call = pl.kernel(
    _agg_es_body,
    out_type=jax.ShapeDtypeStruct((NC, NP, LAT), jnp.float32),
    mesh=_MESH,
    compiler_params=_SC_LINEAR,
    scratch_types=_agg_scratch(KCH),
)


_BR = 1000


def _dinv_block(degt_ref):
    return lax.rsqrt(degt_ref[:, 0] + degt_ref[:, 1] + 1.0)


def _mm_scale_body(x_ref, w_ref, degp_ref, y_ref):
    dinv = _dinv_block(degp_ref)
    y = jnp.dot(x_ref[...], w_ref[...], preferred_element_type=jnp.float32)
    y = y * dinv[:, None]
    y_ref[0] = y[:, :LAT]
    y_ref[1] = y[:, LAT:]


_mm_scale_call = pl.pallas_call(
    _mm_scale_body,
    grid=(N // _BR,),
    in_specs=[
        pl.BlockSpec((_BR, DIN), lambda i: (i, 0)),
        pl.BlockSpec((DIN, HID), lambda i: (0, 0)),
        pl.BlockSpec((_BR, NC), lambda i: (i, 0)),
    ],
    out_specs=pl.BlockSpec((NC, _BR, LAT), lambda i: (0, i, 0)),
    out_shape=jax.ShapeDtypeStruct((NC, NP, LAT), jnp.float32),
)


def _fuse1_body(agg_ref, y1_ref, degp_ref, b1_ref, w2_ref, y2_ref):
    dinv = _dinv_block(degp_ref)
    a = jnp.concatenate([agg_ref[0] + y1_ref[0], agg_ref[1] + y1_ref[1]],
                        axis=1)
    z = jnp.maximum(a * dinv[:, None] + b1_ref[...][None, :], 0.0)
    y2 = jnp.dot(z, w2_ref[...], preferred_element_type=jnp.float32)
    y2_ref[...] = y2 * dinv[:, None]


_fuse1_call = pl.pallas_call(
    _fuse1_body,
    grid=(N // _BR,),
    in_specs=[
        pl.BlockSpec((NC, _BR, LAT), lambda i: (0, i, 0)),
        pl.BlockSpec((NC, _BR, LAT), lambda i: (0, i, 0)),
        pl.BlockSpec((_BR, NC), lambda i: (i, 0)),
        pl.BlockSpec((HID,), lambda i: (0,)),
        pl.BlockSpec((HID, LAT), lambda i: (0, 0)),
    ],
    out_specs=pl.BlockSpec((_BR, LAT), lambda i: (i, 0)),
    out_shape=jax.ShapeDtypeStruct((NP, LAT), jnp.float32),
)


def _fuse2_body(agg_ref, y2_ref, degp_ref, b2_ref, wmu_ref, bmu_ref,
                wlv_ref, blv_ref, mu_ref, lv_ref):
    dinv = _dinv_block(degp_ref)
    a = agg_ref[0] + agg_ref[1] + y2_ref[...]
    z = a * dinv[:, None] + b2_ref[...][None, :]
    mu_ref[...] = (jnp.dot(z, wmu_ref[...], preferred_element_type=jnp.float32)
                   + bmu_ref[...][None, :])
    lv_ref[...] = (jnp.dot(z, wlv_ref[...], preferred_element_type=jnp.float32)
                   + blv_ref[...][None, :])


_fuse2_call = pl.pallas_call(
    _fuse2_body,
    grid=(N // _BR,),
    in_specs=[
        pl.BlockSpec((NC, _BR, LAT), lambda i: (0, i, 0)),
        pl.BlockSpec((_BR, LAT), lambda i: (i, 0)),
        pl.BlockSpec((_BR, NC), lambda i: (i, 0)),
        pl.BlockSpec((LAT,), lambda i: (0,)),
        pl.BlockSpec((LAT, LAT), lambda i: (0, 0)),
        pl.BlockSpec((LAT,), lambda i: (0,)),
        pl.BlockSpec((LAT, LAT), lambda i: (0, 0)),
        pl.BlockSpec((LAT,), lambda i: (0,)),
    ],
    out_specs=[
        pl.BlockSpec((_BR, LAT), lambda i: (i, 0)),
        pl.BlockSpec((_BR, LAT), lambda i: (i, 0)),
    ],
    out_shape=[
        jax.ShapeDtypeStruct((N, LAT), jnp.float32),
        jax.ShapeDtypeStruct((N, LAT), jnp.float32),
    ],
)


def kernel(x, edge_index, W1, b1, W2, b2, W_mu, b_mu, W_lv, b_lv):
    pad_idx = (jnp.arange(EP - E, dtype=jnp.int32) % PAD) + N
    src = jnp.concatenate([edge_index[0], pad_idx])
    dst = jnp.concatenate([edge_index[1], pad_idx])
    srcp16 = src.reshape(NS, KCH1, CH)
    dstp16 = dst.reshape(NS, KCH1, CH)
    srcp32 = src.reshape(NW, KCH, CH)
    dstp32 = dst.reshape(NW, KCH, CH)
    zeros_r = jnp.zeros((RPT,), jnp.float32)
    ones_r = jnp.ones((CH,), jnp.float32)
    zrows = jnp.zeros((CH, LAT), jnp.float32)

    degp = _deg_call(dstp32, zeros_r, ones_r)
    degt = degp.T
    y1cs = _mm_scale_call(x, W1, degt)
    agg1 = _agg_cs_call(y1cs, srcp16, dstp16, zrows)
    y2 = _fuse1_call(agg1, y1cs, degt, b1, W2)
    agg2 = _agg_es_call(y2, srcp32, dstp32, zrows)
    mu, lv = _fuse2_call(agg2, y2, degt, b2, W_mu, b_mu, W_lv, b_lv)
    return (mu, lv)
